# baseline (speedup 1.0000x reference)
import functools
import math

import jax
import jax.numpy as jnp
from jax import lax
from jax.experimental import pallas as pl
from jax.experimental.pallas import tpu as pltpu
from jax.experimental.pallas import tpu_sc as plsc

_TOPK = 2

_NC = 2
_NS = 16
_NW = _NC * _NS
_RB = 128
_EPW = 48
_LMAX = _NW * _EPW


def _router_body(e, x_ref, rw_ref, tril_ref,
                 e0_ref, e1_ref, r0_ref, r1_ref, w0_ref, w1_ref, cnt_ref,
                 carry_ref):
    pid = pl.program_id(0)

    @pl.when(pid == 0)
    def _():
        carry_ref[...] = jnp.zeros_like(carry_ref)

    xb = x_ref[...]
    logits = jnp.dot(xb.astype(jnp.bfloat16), rw_ref[...].astype(jnp.bfloat16),
                     preferred_element_type=jnp.float32)
    m = jnp.max(logits, axis=-1, keepdims=True)
    p = jnp.exp(logits - m)
    p = p / jnp.sum(p, axis=-1, keepdims=True)

    eidx = lax.broadcasted_iota(jnp.int32, p.shape, 1)
    v0 = jnp.max(p, axis=-1, keepdims=True)
    i0 = jnp.min(jnp.where(p == v0, eidx, e), axis=-1, keepdims=True)
    p2 = jnp.where(eidx == i0, -1.0, p)
    v1 = jnp.max(p2, axis=-1, keepdims=True)
    i1 = jnp.min(jnp.where(p2 == v1, eidx, e), axis=-1, keepdims=True)
    s = v0 + v1
    w0 = v0 / s
    w1 = v1 / s

    oh0 = (eidx == i0).astype(jnp.float32)
    oh1 = (eidx == i1).astype(jnp.float32)
    oh = oh0 + oh1
    strict = jnp.dot(tril_ref[...], oh.astype(jnp.bfloat16),
                     preferred_element_type=jnp.float32)
    posf = carry_ref[0:1, 0:e] + strict
    r0 = jnp.sum(oh0 * posf, axis=-1, keepdims=True)
    r1 = jnp.sum(oh1 * posf, axis=-1, keepdims=True)

    e0_ref[0] = i0
    e1_ref[0] = i1
    r0_ref[0] = r0
    r1_ref[0] = r1
    w0_ref[0] = w0
    w1_ref[0] = w1
    carry_ref[0:1, 0:e] = carry_ref[0:1, 0:e] + jnp.sum(oh, axis=0,
                                                        keepdims=True)
    cnt_ref[...] = carry_ref[...]


def _run_router(x, router_w):
    a, d = x.shape
    e = router_w.shape[1]
    bt = min(1024, a)
    nb = a // bt
    tril = jnp.tril(jnp.ones((bt, bt), jnp.bfloat16), k=-1)
    out_shapes = (
        jax.ShapeDtypeStruct((nb, bt, 1), jnp.int32),
        jax.ShapeDtypeStruct((nb, bt, 1), jnp.int32),
        jax.ShapeDtypeStruct((nb, bt, 1), jnp.float32),
        jax.ShapeDtypeStruct((nb, bt, 1), jnp.float32),
        jax.ShapeDtypeStruct((nb, bt, 1), jnp.float32),
        jax.ShapeDtypeStruct((nb, bt, 1), jnp.float32),
        jax.ShapeDtypeStruct((8, 128), jnp.float32),
    )
    tok_spec = pl.BlockSpec((1, bt, 1), lambda i: (i, 0, 0))
    return pl.pallas_call(
        functools.partial(_router_body, e),
        grid=(nb,),
        in_specs=[
            pl.BlockSpec((bt, d), lambda i: (i, 0)),
            pl.BlockSpec((d, e), lambda i: (0, 0)),
            pl.BlockSpec((bt, bt), lambda i: (0, 0)),
        ],
        out_specs=[tok_spec, tok_spec, tok_spec, tok_spec, tok_spec,
                   tok_spec, pl.BlockSpec((8, 128), lambda i: (0, 0))],
        out_shape=out_shapes,
        scratch_shapes=[pltpu.VMEM((8, 128), jnp.float32)],
        compiler_params=pltpu.CompilerParams(
            dimension_semantics=("arbitrary",)),
    )(x, router_w, tril)


def _pos_body(e, cap, cap2, bt, ytrash, atr,
              e0_ref, e1_ref, r0_ref, r1_ref, w0_ref, w1_ref, cnt_ref,
              tril_ref,
              p0_ref, p1_ref, sm0_ref, sm1_ref,
              wr0_ref, wr1_ref, blk_ref, carry_ref):
    pid = pl.program_id(0)

    @pl.when(pid == 0)
    def _():
        carry_ref[...] = jnp.zeros_like(carry_ref)

    counts = cnt_ref[0:1, 0:e]
    adj = counts - (counts > 0).astype(jnp.float32)
    capm1 = jnp.float32(cap - 1)
    capf = jnp.float32(cap)
    cap2f = jnp.float32(cap2)

    adjb = jnp.broadcast_to(adj, (e, e))
    cntb = jnp.broadcast_to(counts, (e, e))
    sub = lax.broadcasted_iota(jnp.int32, (e, e), 0)
    lane = lax.broadcasted_iota(jnp.int32, (e, e), 1)
    offc = jnp.sum(adjb * (lane < sub).astype(jnp.float32), axis=-1,
                   keepdims=True)
    cntc = jnp.sum(cntb * (lane == sub).astype(jnp.float32), axis=-1,
                   keepdims=True)
    startc = jnp.minimum(offc, capm1)
    endc = jnp.where(cntc > 0.0, jnp.minimum(offc + cntc, capf), 0.0)
    livec = jnp.maximum(endc - startc, 0.0)
    nlive = jnp.sum(livec, axis=0, keepdims=True)
    rbf = jnp.float32(_RB)
    lo = jnp.floor(startc / rbf)
    hi = jnp.floor((endc + rbf - 1.0) / rbf)
    lane128 = lax.broadcasted_iota(jnp.int32, (e, 128), 1)
    vals = jnp.where(lane128 == 0, lo, jnp.where(lane128 == 1, hi, 0.0))
    blk_ref[...] = vals.astype(jnp.int32)

    def one(e_ref, r_ref, w_ref):
        ei = e_ref[0]
        r = r_ref[0]
        w = w_ref[0]
        eidx = lax.broadcasted_iota(jnp.int32, (bt, e), 1)
        lt = (eidx < ei).astype(jnp.float32)
        off = jnp.sum(adj * lt, axis=-1, keepdims=True)
        eeq = (eidx == ei).astype(jnp.float32)
        cnt_e = jnp.sum(eeq * counts, axis=-1, keepdims=True)
        posraw = r + off
        keep = ((posraw < capm1) | (r == cnt_e - 1.0)).astype(jnp.float32)
        pos = jnp.minimum(posraw, capm1)
        ebase = ei.astype(jnp.float32) * cap2f
        slot = ebase + pos
        trash = ebase + capf
        return keep, slot, trash, jnp.where(keep > 0.0, w, 0.0)

    keep0, slot0, trash0, weff0 = one(e0_ref, r0_ref, w0_ref)
    keep1, slot1, trash1, weff1 = one(e1_ref, r1_ref, w1_ref)

    ktot = keep0 + keep1
    kstrict = jnp.dot(tril_ref[...], ktot.astype(jnp.bfloat16),
                      preferred_element_type=jnp.float32)
    carryk = carry_ref[0:1, 0:1]
    kpre0 = carryk + kstrict
    kpre1 = kpre0 + keep0
    tg = (lax.broadcasted_iota(jnp.int32, (bt, 1), 0)
          + pid * bt).astype(jnp.float32)
    lrank0 = 2.0 * tg - kpre0
    lrank1 = 2.0 * tg + 1.0 - kpre1
    lmaxm1 = jnp.float32(_LMAX - 1)
    p0 = jnp.where(keep0 > 0.0, kpre0, jnp.minimum(nlive + lrank0, lmaxm1))
    p1 = jnp.where(keep1 > 0.0, kpre1, jnp.minimum(nlive + lrank1, lmaxm1))
    ytr = jnp.float32(ytrash)
    ym0 = jnp.where(keep0 > 0.0, tg, ytr)
    ym1 = jnp.where(keep1 > 0.0, jnp.float32(atr) + tg, ytr)
    smv0 = jnp.where(keep0 > 0.0, slot0, trash0)
    smv1 = jnp.where(keep1 > 0.0, slot1, trash1)
    lane_m = lax.broadcasted_iota(jnp.int32, (bt, 128), 1)

    def meta_rows(smv, ymv):
        m = jnp.where(lane_m == 0, tg,
                      jnp.where(lane_m == 1, smv,
                                jnp.where(lane_m == 2, ymv, 0.0)))
        return m.astype(jnp.int32)

    p0_ref[0] = p0.astype(jnp.int32)
    p1_ref[0] = p1.astype(jnp.int32)
    sm0_ref[0] = meta_rows(smv0, ym0)
    sm1_ref[0] = meta_rows(smv1, ym1)
    wr0_ref[0] = weff0
    wr1_ref[0] = weff1
    carry_ref[0:1, 0:1] = carryk + jnp.sum(ktot, axis=0, keepdims=True)


def _run_pos(e0, e1, r0, r1, w0, w1, counts, e, cap, cap2, ytrash, atr):
    nb, bt, _ = e0.shape
    tril = jnp.tril(jnp.ones((bt, bt), jnp.bfloat16), k=-1)
    tok_spec = pl.BlockSpec((1, bt, 1), lambda i: (i, 0, 0))
    meta_spec = pl.BlockSpec((1, bt, 128), lambda i: (i, 0, 0))
    ispec = [tok_spec] * 6 + [pl.BlockSpec((8, 128), lambda i: (0, 0)),
                              pl.BlockSpec((bt, bt), lambda i: (0, 0))]
    tok_i32 = jax.ShapeDtypeStruct((nb, bt, 1), jnp.int32)
    tok_f32 = jax.ShapeDtypeStruct((nb, bt, 1), jnp.float32)
    meta_i32 = jax.ShapeDtypeStruct((nb, bt, 128), jnp.int32)
    return pl.pallas_call(
        functools.partial(_pos_body, e, cap, cap2, bt, ytrash, atr),
        grid=(nb,),
        in_specs=ispec,
        out_specs=[tok_spec, tok_spec, meta_spec, meta_spec, tok_spec,
                   tok_spec, pl.BlockSpec((e, 128), lambda i: (0, 0))],
        out_shape=(tok_i32, tok_i32, meta_i32, meta_i32,
                   tok_f32, tok_f32,
                   jax.ShapeDtypeStruct((e, 128), jnp.int32)),
        scratch_shapes=[pltpu.VMEM((8, 128), jnp.float32)],
        compiler_params=pltpu.CompilerParams(
            dimension_semantics=("arbitrary",)),
    )(e0, e1, r0, r1, w0, w1, counts, tril)


def _make_listbuild(a):
    n_tok = a // _NW
    nch = n_tok // 64
    mesh = plsc.VectorSubcoreMesh(core_axis_name="c", subcore_axis_name="s")

    @functools.partial(
        pl.kernel, mesh=mesh,
        out_type=jax.ShapeDtypeStruct((_LMAX, 128), jnp.int32),
        scratch_types=[
            pltpu.VMEM((nch, 64), jnp.int32),
            pltpu.VMEM((64, 128), jnp.int32),
            pltpu.VMEM((64, 128), jnp.int32),
            pltpu.SemaphoreType.DMA,
            pltpu.SemaphoreType.DMA,
            pltpu.SemaphoreType.DMA,
            pltpu.SemaphoreType.DMA,
        ],
    )
    def listbuild(p0_hbm, p1_hbm, m0_hbm, m1_hbm,
                  list_hbm, p_v, rowa, rowb, sla, slb, sea, seb):
        wid = lax.axis_index("s") * _NC + lax.axis_index("c")
        rows = (rowa, rowb)
        sl = (sla, slb)
        sems = (sea, seb)
        for k, (pk, mk) in enumerate(((p0_hbm, m0_hbm), (p1_hbm, m1_hbm))):
            pltpu.sync_copy(pk.at[wid], p_v)
            pltpu.async_copy(mk.at[wid, 0], rows[0], sl[0])
            if nch > 1:
                pltpu.async_copy(mk.at[wid, 1], rows[1], sl[1])
            for ch in range(nch):
                b = ch % 2
                pltpu.make_async_copy(mk.at[wid, ch], rows[b], sl[b]).wait()
                pltpu.async_copy(rows[b], list_hbm.at[p_v.at[ch]], sems[b])
                pltpu.make_async_copy(rows[b], list_hbm.at[p_v.at[ch]],
                                      sems[b]).wait()
                if ch + 2 < nch:
                    pltpu.async_copy(mk.at[wid, ch + 2], rows[b], sl[b])

    return listbuild


def _run_extract(clist):
    lm = clist.shape[0]

    def body(cl_ref, tok_ref, slot_ref, ydst_ref):
        blkv = cl_ref[...]
        tok_ref[...] = blkv[:, 0:1]
        slot_ref[...] = blkv[:, 1:2]
        ydst_ref[...] = blkv[:, 2:3]

    col = jax.ShapeDtypeStruct((lm, 1), jnp.int32)
    return pl.pallas_call(
        body,
        grid=(1,),
        in_specs=[pl.BlockSpec((lm, 128), lambda i: (0, 0))],
        out_specs=[pl.BlockSpec((lm, 1), lambda i: (0, 0))] * 3,
        out_shape=(col, col, col),
    )(clist)


def _make_mover(src_rows, dst_rows, d):
    mesh = plsc.VectorSubcoreMesh(core_axis_name="c", subcore_axis_name="s")
    ng = _EPW // _NS

    @functools.partial(
        pl.kernel, mesh=mesh,
        out_type=jax.ShapeDtypeStruct((dst_rows, d), jnp.float32),
        scratch_types=[
            pltpu.VMEM((ng, _NS), jnp.int32),
            pltpu.VMEM((ng, _NS), jnp.int32),
            pltpu.VMEM((_NS, d), jnp.float32),
            pltpu.VMEM((_NS, d), jnp.float32),
            pltpu.SemaphoreType.DMA,
            pltpu.SemaphoreType.DMA,
            pltpu.SemaphoreType.DMA,
            pltpu.SemaphoreType.DMA,
        ],
    )
    def mover(si_hbm, di_hbm, src_hbm, dst_hbm, si_v, di_v,
              bufa, bufb, sga, sgb, ssa, ssb):
        wid = lax.axis_index("s") * _NC + lax.axis_index("c")
        pltpu.sync_copy(si_hbm.at[wid], si_v)
        pltpu.sync_copy(di_hbm.at[wid], di_v)
        bufs = (bufa, bufb)
        sg = (sga, sgb)
        ss = (ssa, ssb)
        for g in range(min(2, ng)):
            pltpu.async_copy(src_hbm.at[si_v.at[g]], bufs[g], sg[g])
        for g in range(ng):
            b = g % 2
            pltpu.make_async_copy(src_hbm.at[si_v.at[g]], bufs[b],
                                  sg[b]).wait()
            pltpu.async_copy(bufs[b], dst_hbm.at[di_v.at[g]], ss[b])
            pltpu.make_async_copy(bufs[b], dst_hbm.at[di_v.at[g]],
                                  ss[b]).wait()
            if g + 2 < ng:
                pltpu.async_copy(src_hbm.at[si_v.at[g + 2]], bufs[b], sg[b])

    return mover


def _mlp_body(e, nh, sinfo_ref, xb_ref, w1_ref, w3_ref, w2_ref, out_ref,
              xbf_ref):
    ei = pl.program_id(0)
    hstep = pl.program_id(1)
    lo = sinfo_ref[ei]
    hi = sinfo_ref[e + ei]
    w1b = w1_ref[0].astype(jnp.bfloat16)
    w3b = w3_ref[0].astype(jnp.bfloat16)
    w2b = w2_ref[0].astype(jnp.bfloat16)

    @pl.when(hstep == 0)
    def _():
        def cast_body(i, carry):
            rs = pl.ds(i * _RB, _RB)
            xbf_ref[rs, :] = xb_ref[0, rs, :].astype(jnp.bfloat16)
            return carry
        lax.fori_loop(lo, hi, cast_body, 0)

    def blk_body(i, carry):
        rs = pl.ds(i * _RB, _RB)
        xr = xbf_ref[rs, :]
        aa = jnp.dot(xr, w1b, preferred_element_type=jnp.float32)
        bb = jnp.dot(xr, w3b, preferred_element_type=jnp.float32)
        g = aa * jax.nn.sigmoid(aa) * bb
        cc = jnp.dot(g.astype(jnp.bfloat16), w2b,
                     preferred_element_type=jnp.float32)
        prev = out_ref[0, rs, :]
        out_ref[0, rs, :] = jnp.where(hstep == 0, cc, prev + cc)
        return carry

    lax.fori_loop(lo, hi, blk_body, 0)


def _run_mlp(xbuf, w1, w3, w2, sinfo):
    e, cap2, d = xbuf.shape
    h = w1.shape[2]
    bh = 256 if h % 256 == 0 else h
    nh = h // bh
    grid_spec = pltpu.PrefetchScalarGridSpec(
        num_scalar_prefetch=1,
        grid=(e, nh),
        in_specs=[
            pl.BlockSpec((1, cap2, d), lambda i, j, *_: (i, 0, 0)),
            pl.BlockSpec((1, d, bh), lambda i, j, *_: (i, 0, j)),
            pl.BlockSpec((1, d, bh), lambda i, j, *_: (i, 0, j)),
            pl.BlockSpec((1, bh, d), lambda i, j, *_: (i, j, 0)),
        ],
        out_specs=pl.BlockSpec((1, cap2, d), lambda i, j, *_: (i, 0, 0)),
        scratch_shapes=[pltpu.VMEM((cap2, d), jnp.bfloat16)],
    )
    return pl.pallas_call(
        functools.partial(_mlp_body, e, nh),
        grid_spec=grid_spec,
        out_shape=jax.ShapeDtypeStruct((e, cap2, d), jnp.float32),
        compiler_params=pltpu.CompilerParams(
            dimension_semantics=("arbitrary", "arbitrary")),
    )(sinfo, xbuf, w1, w3, w2)


def _fin_body(ya_ref, yb_ref, w0_ref, w1_ref, y_ref):
    w0 = w0_ref[0]
    w1 = w1_ref[0]
    ca = jnp.where(w0 > 0.0, w0 * ya_ref[...], 0.0)
    cb = jnp.where(w1 > 0.0, w1 * yb_ref[...], 0.0)
    y_ref[...] = ca + cb


def _run_final(yab, wr0, wr1, a, d, atr):
    nb, bt, _ = wr0.shape
    off = atr // bt
    tok_spec = pl.BlockSpec((1, bt, 1), lambda i: (i, 0, 0))
    return pl.pallas_call(
        _fin_body,
        grid=(nb,),
        in_specs=[
            pl.BlockSpec((bt, d), lambda i: (i, 0)),
            pl.BlockSpec((bt, d), lambda i: (off + i, 0)),
            tok_spec, tok_spec,
        ],
        out_specs=pl.BlockSpec((bt, d), lambda i: (i, 0)),
        out_shape=jax.ShapeDtypeStruct((a, d), jnp.float32),
    )(yab, yab, wr0, wr1)


def kernel(x, router_w, w1, w2, w3):
    a, d = x.shape
    e = router_w.shape[1]
    cap = max(1, int(math.ceil(1.25 * a * _TOPK / e)))
    cap2 = cap + 8
    n_slots = e * cap2
    atr = a + 1024
    ryab = 2 * atr
    ytrash = ryab - 1024

    e0, e1, r0, r1, w0, w1r_, counts = _run_router(x, router_w)
    p0, p1, m0, m1, wr0, wr1, blk = _run_pos(
        e0, e1, r0, r1, w0, w1r_, counts, e, cap, cap2, ytrash, atr)
    sinfo = jnp.concatenate([blk[:, 0], blk[:, 1]])

    n_tok = a // _NW
    nch = n_tok // 64
    clist = _make_listbuild(a)(
        p0.reshape(_NW, nch, 64), p1.reshape(_NW, nch, 64),
        m0.reshape(_NW, nch, 64, 128), m1.reshape(_NW, nch, 64, 128))

    ltok, lslot, lydst = _run_extract(clist)
    ng = _EPW // _NS
    ltok = ltok.reshape(_NW, ng, _NS)
    lslot = lslot.reshape(_NW, ng, _NS)
    lydst = lydst.reshape(_NW, ng, _NS)
    xbuf = _make_mover(a, n_slots, d)(ltok, lslot, x)

    yslots = _run_mlp(xbuf.reshape(e, cap2, d), w1, w3, w2, sinfo)

    yab = _make_mover(n_slots, ryab, d)(
        lslot, lydst, yslots.reshape(n_slots, d))

    return _run_final(yab, wr0, wr1, a, d, atr)


# speedup vs baseline: 1.8386x; 1.8386x over previous
import functools
import math

import jax
import jax.numpy as jnp
from jax import lax
from jax.experimental import pallas as pl
from jax.experimental.pallas import tpu as pltpu
from jax.experimental.pallas import tpu_sc as plsc

_TOPK = 2

_NC = 2
_NS = 16
_NW = _NC * _NS
_RB = 128
_EPW = 48
_LMAX = _NW * _EPW


def _router_body(e, x_ref, rw_ref, tril_ref,
                 e0_ref, e1_ref, r0_ref, r1_ref, w0_ref, w1_ref, cnt_ref,
                 carry_ref):
    pid = pl.program_id(0)

    @pl.when(pid == 0)
    def _():
        carry_ref[...] = jnp.zeros_like(carry_ref)

    xb = x_ref[...]
    logits = jnp.dot(xb.astype(jnp.bfloat16), rw_ref[...].astype(jnp.bfloat16),
                     preferred_element_type=jnp.float32)
    m = jnp.max(logits, axis=-1, keepdims=True)
    p = jnp.exp(logits - m)
    p = p / jnp.sum(p, axis=-1, keepdims=True)

    eidx = lax.broadcasted_iota(jnp.int32, p.shape, 1)
    v0 = jnp.max(p, axis=-1, keepdims=True)
    i0 = jnp.min(jnp.where(p == v0, eidx, e), axis=-1, keepdims=True)
    p2 = jnp.where(eidx == i0, -1.0, p)
    v1 = jnp.max(p2, axis=-1, keepdims=True)
    i1 = jnp.min(jnp.where(p2 == v1, eidx, e), axis=-1, keepdims=True)
    s = v0 + v1
    w0 = v0 / s
    w1 = v1 / s

    oh0 = (eidx == i0).astype(jnp.float32)
    oh1 = (eidx == i1).astype(jnp.float32)
    oh = oh0 + oh1
    strict = jnp.dot(tril_ref[...], oh.astype(jnp.bfloat16),
                     preferred_element_type=jnp.float32)
    posf = carry_ref[0:1, 0:e] + strict
    r0 = jnp.sum(oh0 * posf, axis=-1, keepdims=True)
    r1 = jnp.sum(oh1 * posf, axis=-1, keepdims=True)

    e0_ref[0] = i0
    e1_ref[0] = i1
    r0_ref[0] = r0
    r1_ref[0] = r1
    w0_ref[0] = w0
    w1_ref[0] = w1
    carry_ref[0:1, 0:e] = carry_ref[0:1, 0:e] + jnp.sum(oh, axis=0,
                                                        keepdims=True)
    cnt_ref[...] = carry_ref[...]


def _run_router(x, router_w):
    a, d = x.shape
    e = router_w.shape[1]
    bt = min(1024, a)
    nb = a // bt
    tril = jnp.tril(jnp.ones((bt, bt), jnp.bfloat16), k=-1)
    out_shapes = (
        jax.ShapeDtypeStruct((nb, bt, 1), jnp.int32),
        jax.ShapeDtypeStruct((nb, bt, 1), jnp.int32),
        jax.ShapeDtypeStruct((nb, bt, 1), jnp.float32),
        jax.ShapeDtypeStruct((nb, bt, 1), jnp.float32),
        jax.ShapeDtypeStruct((nb, bt, 1), jnp.float32),
        jax.ShapeDtypeStruct((nb, bt, 1), jnp.float32),
        jax.ShapeDtypeStruct((8, 128), jnp.float32),
    )
    tok_spec = pl.BlockSpec((1, bt, 1), lambda i: (i, 0, 0))
    return pl.pallas_call(
        functools.partial(_router_body, e),
        grid=(nb,),
        in_specs=[
            pl.BlockSpec((bt, d), lambda i: (i, 0)),
            pl.BlockSpec((d, e), lambda i: (0, 0)),
            pl.BlockSpec((bt, bt), lambda i: (0, 0)),
        ],
        out_specs=[tok_spec, tok_spec, tok_spec, tok_spec, tok_spec,
                   tok_spec, pl.BlockSpec((8, 128), lambda i: (0, 0))],
        out_shape=out_shapes,
        scratch_shapes=[pltpu.VMEM((8, 128), jnp.float32)],
        compiler_params=pltpu.CompilerParams(
            dimension_semantics=("arbitrary",)),
    )(x, router_w, tril)


def _pos_body(e, cap, cap2, bt, ytrash, atr,
              e0_ref, e1_ref, r0_ref, r1_ref, w0_ref, w1_ref, cnt_ref,
              tril_ref,
              p0_ref, p1_ref, sm0_ref, sm1_ref,
              wr0_ref, wr1_ref, blk_ref, carry_ref):
    pid = pl.program_id(0)

    @pl.when(pid == 0)
    def _():
        carry_ref[...] = jnp.zeros_like(carry_ref)

    counts = cnt_ref[0:1, 0:e]
    adj = counts - (counts > 0).astype(jnp.float32)
    capm1 = jnp.float32(cap - 1)
    capf = jnp.float32(cap)
    cap2f = jnp.float32(cap2)

    adjb = jnp.broadcast_to(adj, (e, e))
    cntb = jnp.broadcast_to(counts, (e, e))
    sub = lax.broadcasted_iota(jnp.int32, (e, e), 0)
    lane = lax.broadcasted_iota(jnp.int32, (e, e), 1)
    offc = jnp.sum(adjb * (lane < sub).astype(jnp.float32), axis=-1,
                   keepdims=True)
    cntc = jnp.sum(cntb * (lane == sub).astype(jnp.float32), axis=-1,
                   keepdims=True)
    startc = jnp.minimum(offc, capm1)
    endc = jnp.where(cntc > 0.0, jnp.minimum(offc + cntc, capf), 0.0)
    livec = jnp.maximum(endc - startc, 0.0)
    nlive = jnp.sum(livec, axis=0, keepdims=True)
    rbf = jnp.float32(_RB)
    lo = jnp.floor(startc / rbf)
    hi = jnp.floor((endc + rbf - 1.0) / rbf)
    lane128 = lax.broadcasted_iota(jnp.int32, (e, 128), 1)
    vals = jnp.where(lane128 == 0, lo, jnp.where(lane128 == 1, hi, 0.0))
    blk_ref[...] = vals.astype(jnp.int32)

    def one(e_ref, r_ref, w_ref):
        ei = e_ref[0]
        r = r_ref[0]
        w = w_ref[0]
        eidx = lax.broadcasted_iota(jnp.int32, (bt, e), 1)
        lt = (eidx < ei).astype(jnp.float32)
        off = jnp.sum(adj * lt, axis=-1, keepdims=True)
        eeq = (eidx == ei).astype(jnp.float32)
        cnt_e = jnp.sum(eeq * counts, axis=-1, keepdims=True)
        posraw = r + off
        keep = ((posraw < capm1) | (r == cnt_e - 1.0)).astype(jnp.float32)
        pos = jnp.minimum(posraw, capm1)
        ebase = ei.astype(jnp.float32) * cap2f
        slot = ebase + pos
        trash = ebase + capf
        return keep, slot, trash, jnp.where(keep > 0.0, w, 0.0)

    keep0, slot0, trash0, weff0 = one(e0_ref, r0_ref, w0_ref)
    keep1, slot1, trash1, weff1 = one(e1_ref, r1_ref, w1_ref)

    ktot = keep0 + keep1
    kstrict = jnp.dot(tril_ref[...], ktot.astype(jnp.bfloat16),
                      preferred_element_type=jnp.float32)
    carryk = carry_ref[0:1, 0:1]
    kpre0 = carryk + kstrict
    kpre1 = kpre0 + keep0
    tgi = lax.broadcasted_iota(jnp.int32, (bt, 1), 0) + pid * bt
    tg = tgi.astype(jnp.float32)
    nlive_i = nlive.astype(jnp.int32)
    tail = _LMAX - nlive_i
    lr0 = 2 * tgi - kpre0.astype(jnp.int32)
    lr1 = 2 * tgi + 1 - kpre1.astype(jnp.int32)
    p0 = jnp.where(keep0 > 0.0, kpre0.astype(jnp.int32),
                   nlive_i + lax.rem(lr0, tail))
    p1 = jnp.where(keep1 > 0.0, kpre1.astype(jnp.int32),
                   nlive_i + lax.rem(lr1, tail))
    ytr = ytrash + lax.rem(tgi, 1024)
    ym0 = jnp.where(keep0 > 0.0, tgi, ytr)
    ym1 = jnp.where(keep1 > 0.0, atr + tgi, ytr)
    trash_i = trash0.astype(jnp.int32) + lax.rem(tgi, 8)
    trash_j = trash1.astype(jnp.int32) + lax.rem(tgi, 8)
    smv0 = jnp.where(keep0 > 0.0, slot0.astype(jnp.int32), trash_i)
    smv1 = jnp.where(keep1 > 0.0, slot1.astype(jnp.int32), trash_j)
    lane_m = lax.broadcasted_iota(jnp.int32, (bt, 128), 1)

    def meta_rows(smv, ymv):
        return jnp.where(lane_m == 0, tgi,
                         jnp.where(lane_m == 1, smv,
                                   jnp.where(lane_m == 2, ymv, 0)))

    p0_ref[0] = p0
    p1_ref[0] = p1
    sm0_ref[0] = meta_rows(smv0, ym0)
    sm1_ref[0] = meta_rows(smv1, ym1)
    wr0_ref[0] = weff0
    wr1_ref[0] = weff1
    carry_ref[0:1, 0:1] = carryk + jnp.sum(ktot, axis=0, keepdims=True)


def _run_pos(e0, e1, r0, r1, w0, w1, counts, e, cap, cap2, ytrash, atr):
    nb, bt, _ = e0.shape
    tril = jnp.tril(jnp.ones((bt, bt), jnp.bfloat16), k=-1)
    tok_spec = pl.BlockSpec((1, bt, 1), lambda i: (i, 0, 0))
    meta_spec = pl.BlockSpec((1, bt, 128), lambda i: (i, 0, 0))
    ispec = [tok_spec] * 6 + [pl.BlockSpec((8, 128), lambda i: (0, 0)),
                              pl.BlockSpec((bt, bt), lambda i: (0, 0))]
    tok_i32 = jax.ShapeDtypeStruct((nb, bt, 1), jnp.int32)
    tok_f32 = jax.ShapeDtypeStruct((nb, bt, 1), jnp.float32)
    meta_i32 = jax.ShapeDtypeStruct((nb, bt, 128), jnp.int32)
    return pl.pallas_call(
        functools.partial(_pos_body, e, cap, cap2, bt, ytrash, atr),
        grid=(nb,),
        in_specs=ispec,
        out_specs=[tok_spec, tok_spec, meta_spec, meta_spec, tok_spec,
                   tok_spec, pl.BlockSpec((e, 128), lambda i: (0, 0))],
        out_shape=(tok_i32, tok_i32, meta_i32, meta_i32,
                   tok_f32, tok_f32,
                   jax.ShapeDtypeStruct((e, 128), jnp.int32)),
        scratch_shapes=[pltpu.VMEM((8, 128), jnp.float32)],
        compiler_params=pltpu.CompilerParams(
            dimension_semantics=("arbitrary",)),
    )(e0, e1, r0, r1, w0, w1, counts, tril)


def _make_listbuild(a):
    n_tok = a // _NW
    nch = n_tok // 64
    mesh = plsc.VectorSubcoreMesh(core_axis_name="c", subcore_axis_name="s")

    @functools.partial(
        pl.kernel, mesh=mesh,
        out_type=jax.ShapeDtypeStruct((_LMAX, 128), jnp.int32),
        scratch_types=[
            pltpu.VMEM((nch, 64), jnp.int32),
            pltpu.VMEM((64, 128), jnp.int32),
            pltpu.VMEM((64, 128), jnp.int32),
            pltpu.SemaphoreType.DMA,
            pltpu.SemaphoreType.DMA,
            pltpu.SemaphoreType.DMA,
            pltpu.SemaphoreType.DMA,
        ],
    )
    def listbuild(p0_hbm, p1_hbm, m0_hbm, m1_hbm,
                  list_hbm, p_v, rowa, rowb, sla, slb, sea, seb):
        wid = lax.axis_index("s") * _NC + lax.axis_index("c")
        rows = (rowa, rowb)
        sl = (sla, slb)
        sems = (sea, seb)
        for k, (pk, mk) in enumerate(((p0_hbm, m0_hbm), (p1_hbm, m1_hbm))):
            pltpu.sync_copy(pk.at[wid], p_v)
            pltpu.async_copy(mk.at[wid, 0], rows[0], sl[0])
            if nch > 1:
                pltpu.async_copy(mk.at[wid, 1], rows[1], sl[1])
            for ch in range(nch):
                b = ch % 2
                pltpu.make_async_copy(mk.at[wid, ch], rows[b], sl[b]).wait()
                pltpu.async_copy(rows[b], list_hbm.at[p_v.at[ch]], sems[b])
                pltpu.make_async_copy(rows[b], list_hbm.at[p_v.at[ch]],
                                      sems[b]).wait()
                if ch + 2 < nch:
                    pltpu.async_copy(mk.at[wid, ch + 2], rows[b], sl[b])

    return listbuild


def _run_extract(clist):
    lm = clist.shape[0]

    def body(cl_ref, tok_ref, slot_ref, ydst_ref):
        blkv = cl_ref[...]
        tok_ref[...] = blkv[:, 0:1]
        slot_ref[...] = blkv[:, 1:2]
        ydst_ref[...] = blkv[:, 2:3]

    col = jax.ShapeDtypeStruct((lm, 1), jnp.int32)
    return pl.pallas_call(
        body,
        grid=(1,),
        in_specs=[pl.BlockSpec((lm, 128), lambda i: (0, 0))],
        out_specs=[pl.BlockSpec((lm, 1), lambda i: (0, 0))] * 3,
        out_shape=(col, col, col),
    )(clist)


def _make_mover(src_rows, dst_rows, d):
    mesh = plsc.VectorSubcoreMesh(core_axis_name="c", subcore_axis_name="s")
    ng = _EPW // _NS

    @functools.partial(
        pl.kernel, mesh=mesh,
        out_type=jax.ShapeDtypeStruct((dst_rows, d), jnp.float32),
        scratch_types=[
            pltpu.VMEM((ng, _NS), jnp.int32),
            pltpu.VMEM((ng, _NS), jnp.int32),
            pltpu.VMEM((_NS, d), jnp.float32),
            pltpu.VMEM((_NS, d), jnp.float32),
            pltpu.SemaphoreType.DMA,
            pltpu.SemaphoreType.DMA,
            pltpu.SemaphoreType.DMA,
            pltpu.SemaphoreType.DMA,
        ],
    )
    def mover(si_hbm, di_hbm, src_hbm, dst_hbm, si_v, di_v,
              bufa, bufb, sga, sgb, ssa, ssb):
        wid = lax.axis_index("s") * _NC + lax.axis_index("c")
        pltpu.sync_copy(si_hbm.at[wid], si_v)
        pltpu.sync_copy(di_hbm.at[wid], di_v)
        bufs = (bufa, bufb)
        sg = (sga, sgb)
        ss = (ssa, ssb)
        for g in range(min(2, ng)):
            pltpu.async_copy(src_hbm.at[si_v.at[g]], bufs[g], sg[g])
        for g in range(ng):
            b = g % 2
            pltpu.make_async_copy(src_hbm.at[si_v.at[g]], bufs[b],
                                  sg[b]).wait()
            pltpu.async_copy(bufs[b], dst_hbm.at[di_v.at[g]], ss[b])
            pltpu.make_async_copy(bufs[b], dst_hbm.at[di_v.at[g]],
                                  ss[b]).wait()
            if g + 2 < ng:
                pltpu.async_copy(src_hbm.at[si_v.at[g + 2]], bufs[b], sg[b])

    return mover


def _mlp_body(e, nh, sinfo_ref, xb_ref, w1_ref, w3_ref, w2_ref, out_ref,
              xbf_ref):
    ei = pl.program_id(0)
    hstep = pl.program_id(1)
    lo = sinfo_ref[ei]
    hi = sinfo_ref[e + ei]
    w1b = w1_ref[0].astype(jnp.bfloat16)
    w3b = w3_ref[0].astype(jnp.bfloat16)
    w2b = w2_ref[0].astype(jnp.bfloat16)

    @pl.when(hstep == 0)
    def _():
        def cast_body(i, carry):
            rs = pl.ds(i * _RB, _RB)
            xbf_ref[rs, :] = xb_ref[0, rs, :].astype(jnp.bfloat16)
            return carry
        lax.fori_loop(lo, hi, cast_body, 0)

    def blk_body(i, carry):
        rs = pl.ds(i * _RB, _RB)
        xr = xbf_ref[rs, :]
        aa = jnp.dot(xr, w1b, preferred_element_type=jnp.float32)
        bb = jnp.dot(xr, w3b, preferred_element_type=jnp.float32)
        g = aa * jax.nn.sigmoid(aa) * bb
        cc = jnp.dot(g.astype(jnp.bfloat16), w2b,
                     preferred_element_type=jnp.float32)
        prev = out_ref[0, rs, :]
        out_ref[0, rs, :] = jnp.where(hstep == 0, cc, prev + cc)
        return carry

    lax.fori_loop(lo, hi, blk_body, 0)


def _run_mlp(xbuf, w1, w3, w2, sinfo):
    e, cap2, d = xbuf.shape
    h = w1.shape[2]
    bh = 256 if h % 256 == 0 else h
    nh = h // bh
    grid_spec = pltpu.PrefetchScalarGridSpec(
        num_scalar_prefetch=1,
        grid=(e, nh),
        in_specs=[
            pl.BlockSpec((1, cap2, d), lambda i, j, *_: (i, 0, 0)),
            pl.BlockSpec((1, d, bh), lambda i, j, *_: (i, 0, j)),
            pl.BlockSpec((1, d, bh), lambda i, j, *_: (i, 0, j)),
            pl.BlockSpec((1, bh, d), lambda i, j, *_: (i, j, 0)),
        ],
        out_specs=pl.BlockSpec((1, cap2, d), lambda i, j, *_: (i, 0, 0)),
        scratch_shapes=[pltpu.VMEM((cap2, d), jnp.bfloat16)],
    )
    return pl.pallas_call(
        functools.partial(_mlp_body, e, nh),
        grid_spec=grid_spec,
        out_shape=jax.ShapeDtypeStruct((e, cap2, d), jnp.float32),
        compiler_params=pltpu.CompilerParams(
            dimension_semantics=("arbitrary", "arbitrary")),
    )(sinfo, xbuf, w1, w3, w2)


def _fin_body(ya_ref, yb_ref, w0_ref, w1_ref, y_ref):
    w0 = w0_ref[0]
    w1 = w1_ref[0]
    ca = jnp.where(w0 > 0.0, w0 * ya_ref[...], 0.0)
    cb = jnp.where(w1 > 0.0, w1 * yb_ref[...], 0.0)
    y_ref[...] = ca + cb


def _run_final(yab, wr0, wr1, a, d, atr):
    nb, bt, _ = wr0.shape
    off = atr // bt
    tok_spec = pl.BlockSpec((1, bt, 1), lambda i: (i, 0, 0))
    return pl.pallas_call(
        _fin_body,
        grid=(nb,),
        in_specs=[
            pl.BlockSpec((bt, d), lambda i: (i, 0)),
            pl.BlockSpec((bt, d), lambda i: (off + i, 0)),
            tok_spec, tok_spec,
        ],
        out_specs=pl.BlockSpec((bt, d), lambda i: (i, 0)),
        out_shape=jax.ShapeDtypeStruct((a, d), jnp.float32),
    )(yab, yab, wr0, wr1)


def kernel(x, router_w, w1, w2, w3):
    a, d = x.shape
    e = router_w.shape[1]
    cap = max(1, int(math.ceil(1.25 * a * _TOPK / e)))
    cap2 = cap + 8
    n_slots = e * cap2
    atr = a + 1024
    ryab = 2 * atr
    ytrash = ryab - 1024

    e0, e1, r0, r1, w0, w1r_, counts = _run_router(x, router_w)
    p0, p1, m0, m1, wr0, wr1, blk = _run_pos(
        e0, e1, r0, r1, w0, w1r_, counts, e, cap, cap2, ytrash, atr)
    sinfo = jnp.concatenate([blk[:, 0], blk[:, 1]])

    n_tok = a // _NW
    nch = n_tok // 64
    clist = _make_listbuild(a)(
        p0.reshape(_NW, nch, 64), p1.reshape(_NW, nch, 64),
        m0.reshape(_NW, nch, 64, 128), m1.reshape(_NW, nch, 64, 128))

    ltok, lslot, lydst = _run_extract(clist)
    ng = _EPW // _NS
    ltok = ltok.reshape(_NW, ng, _NS)
    lslot = lslot.reshape(_NW, ng, _NS)
    lydst = lydst.reshape(_NW, ng, _NS)
    xbuf = _make_mover(a, n_slots, d)(ltok, lslot, x)

    yslots = _run_mlp(xbuf.reshape(e, cap2, d), w1, w3, w2, sinfo)

    yab = _make_mover(n_slots, ryab, d)(
        lslot, lydst, yslots.reshape(n_slots, d))

    return _run_final(yab, wr0, wr1, a, d, atr)


# speedup vs baseline: 2.1589x; 1.1742x over previous
import functools
import math

import jax
import jax.numpy as jnp
from jax import lax
from jax.experimental import pallas as pl
from jax.experimental.pallas import tpu as pltpu
from jax.experimental.pallas import tpu_sc as plsc

_TOPK = 2

_NC = 2
_NS = 16
_NW = _NC * _NS
_RB = 128
_EPW = 48
_LMAX = _NW * _EPW


def _router_body(e, x_ref, rw_ref, tril_ref,
                 e0_ref, e1_ref, r0_ref, r1_ref, w0_ref, w1_ref, cnt_ref,
                 carry_ref):
    pid = pl.program_id(0)

    @pl.when(pid == 0)
    def _():
        carry_ref[...] = jnp.zeros_like(carry_ref)

    xb = x_ref[...]
    logits = jnp.dot(xb.astype(jnp.bfloat16), rw_ref[...].astype(jnp.bfloat16),
                     preferred_element_type=jnp.float32)
    m = jnp.max(logits, axis=-1, keepdims=True)
    p = jnp.exp(logits - m)
    p = p / jnp.sum(p, axis=-1, keepdims=True)

    eidx = lax.broadcasted_iota(jnp.int32, p.shape, 1)
    v0 = jnp.max(p, axis=-1, keepdims=True)
    i0 = jnp.min(jnp.where(p == v0, eidx, e), axis=-1, keepdims=True)
    p2 = jnp.where(eidx == i0, -1.0, p)
    v1 = jnp.max(p2, axis=-1, keepdims=True)
    i1 = jnp.min(jnp.where(p2 == v1, eidx, e), axis=-1, keepdims=True)
    s = v0 + v1
    w0 = v0 / s
    w1 = v1 / s

    oh0 = (eidx == i0).astype(jnp.float32)
    oh1 = (eidx == i1).astype(jnp.float32)
    oh = oh0 + oh1
    strict = jnp.dot(tril_ref[...], oh.astype(jnp.bfloat16),
                     preferred_element_type=jnp.float32)
    posf = carry_ref[0:1, 0:e] + strict
    r0 = jnp.sum(oh0 * posf, axis=-1, keepdims=True)
    r1 = jnp.sum(oh1 * posf, axis=-1, keepdims=True)

    e0_ref[0] = i0
    e1_ref[0] = i1
    r0_ref[0] = r0
    r1_ref[0] = r1
    w0_ref[0] = w0
    w1_ref[0] = w1
    carry_ref[0:1, 0:e] = carry_ref[0:1, 0:e] + jnp.sum(oh, axis=0,
                                                        keepdims=True)
    cnt_ref[...] = carry_ref[...]


def _run_router(x, router_w):
    a, d = x.shape
    e = router_w.shape[1]
    bt = min(1024, a)
    nb = a // bt
    tril = jnp.tril(jnp.ones((bt, bt), jnp.bfloat16), k=-1)
    out_shapes = (
        jax.ShapeDtypeStruct((nb, bt, 1), jnp.int32),
        jax.ShapeDtypeStruct((nb, bt, 1), jnp.int32),
        jax.ShapeDtypeStruct((nb, bt, 1), jnp.float32),
        jax.ShapeDtypeStruct((nb, bt, 1), jnp.float32),
        jax.ShapeDtypeStruct((nb, bt, 1), jnp.float32),
        jax.ShapeDtypeStruct((nb, bt, 1), jnp.float32),
        jax.ShapeDtypeStruct((8, 128), jnp.float32),
    )
    tok_spec = pl.BlockSpec((1, bt, 1), lambda i: (i, 0, 0))
    return pl.pallas_call(
        functools.partial(_router_body, e),
        grid=(nb,),
        in_specs=[
            pl.BlockSpec((bt, d), lambda i: (i, 0)),
            pl.BlockSpec((d, e), lambda i: (0, 0)),
            pl.BlockSpec((bt, bt), lambda i: (0, 0)),
        ],
        out_specs=[tok_spec, tok_spec, tok_spec, tok_spec, tok_spec,
                   tok_spec, pl.BlockSpec((8, 128), lambda i: (0, 0))],
        out_shape=out_shapes,
        scratch_shapes=[pltpu.VMEM((8, 128), jnp.float32)],
        compiler_params=pltpu.CompilerParams(
            dimension_semantics=("arbitrary",)),
    )(x, router_w, tril)


def _pos_body(e, cap, cap2, bt, ytrash, atr,
              e0_ref, e1_ref, r0_ref, r1_ref, w0_ref, w1_ref, cnt_ref,
              tril_ref,
              p0_ref, p1_ref, sm0_ref, sm1_ref,
              wr0_ref, wr1_ref, blk_ref, carry_ref):
    pid = pl.program_id(0)

    @pl.when(pid == 0)
    def _():
        carry_ref[...] = jnp.zeros_like(carry_ref)

    counts = cnt_ref[0:1, 0:e]
    adj = counts - (counts > 0).astype(jnp.float32)
    capm1 = jnp.float32(cap - 1)
    capf = jnp.float32(cap)
    cap2f = jnp.float32(cap2)

    adjb = jnp.broadcast_to(adj, (e, e))
    cntb = jnp.broadcast_to(counts, (e, e))
    sub = lax.broadcasted_iota(jnp.int32, (e, e), 0)
    lane = lax.broadcasted_iota(jnp.int32, (e, e), 1)
    offc = jnp.sum(adjb * (lane < sub).astype(jnp.float32), axis=-1,
                   keepdims=True)
    cntc = jnp.sum(cntb * (lane == sub).astype(jnp.float32), axis=-1,
                   keepdims=True)
    startc = jnp.minimum(offc, capm1)
    endc = jnp.where(cntc > 0.0, jnp.minimum(offc + cntc, capf), 0.0)
    livec = jnp.maximum(endc - startc, 0.0)
    nlive = jnp.sum(livec, axis=0, keepdims=True)
    rbf = jnp.float32(_RB)
    lo = jnp.floor(startc / rbf)
    hi = jnp.floor((endc + rbf - 1.0) / rbf)
    lane128 = lax.broadcasted_iota(jnp.int32, (e, 128), 1)
    vals = jnp.where(lane128 == 0, lo, jnp.where(lane128 == 1, hi, 0.0))
    blk_ref[...] = vals.astype(jnp.int32)

    def one(e_ref, r_ref, w_ref):
        ei = e_ref[0]
        r = r_ref[0]
        w = w_ref[0]
        eidx = lax.broadcasted_iota(jnp.int32, (bt, e), 1)
        lt = (eidx < ei).astype(jnp.float32)
        off = jnp.sum(adj * lt, axis=-1, keepdims=True)
        eeq = (eidx == ei).astype(jnp.float32)
        cnt_e = jnp.sum(eeq * counts, axis=-1, keepdims=True)
        posraw = r + off
        keep = ((posraw < capm1) | (r == cnt_e - 1.0)).astype(jnp.float32)
        pos = jnp.minimum(posraw, capm1)
        ebase = ei.astype(jnp.float32) * cap2f
        slot = ebase + pos
        trash = ebase + capf
        return keep, slot, trash, jnp.where(keep > 0.0, w, 0.0)

    keep0, slot0, trash0, weff0 = one(e0_ref, r0_ref, w0_ref)
    keep1, slot1, trash1, weff1 = one(e1_ref, r1_ref, w1_ref)

    ktot = keep0 + keep1
    kstrict = jnp.dot(tril_ref[...], ktot.astype(jnp.bfloat16),
                      preferred_element_type=jnp.float32)
    carryk = carry_ref[0:1, 0:1]
    kpre0 = carryk + kstrict
    kpre1 = kpre0 + keep0
    tgi = lax.broadcasted_iota(jnp.int32, (bt, 1), 0) + pid * bt
    tg = tgi.astype(jnp.float32)
    nlive_i = nlive.astype(jnp.int32)
    tail = _LMAX - nlive_i
    tail_f = tail.astype(jnp.float32)

    def mod_tail(lr):
        q = jnp.floor(lr.astype(jnp.float32) / tail_f).astype(jnp.int32)
        m = lr - q * tail
        m = jnp.where(m < 0, m + tail, m)
        return jnp.where(m >= tail, m - tail, m)

    lr0 = 2 * tgi - kpre0.astype(jnp.int32)
    lr1 = 2 * tgi + 1 - kpre1.astype(jnp.int32)
    p0 = jnp.where(keep0 > 0.0, kpre0.astype(jnp.int32),
                   nlive_i + mod_tail(lr0))
    p1 = jnp.where(keep1 > 0.0, kpre1.astype(jnp.int32),
                   nlive_i + mod_tail(lr1))
    ytr = ytrash + jnp.bitwise_and(tgi, 1023)
    ym0 = jnp.where(keep0 > 0.0, tgi, ytr)
    ym1 = jnp.where(keep1 > 0.0, atr + tgi, ytr)
    trash_i = trash0.astype(jnp.int32) + jnp.bitwise_and(tgi, 7)
    trash_j = trash1.astype(jnp.int32) + jnp.bitwise_and(tgi, 7)
    smv0 = jnp.where(keep0 > 0.0, slot0.astype(jnp.int32), trash_i)
    smv1 = jnp.where(keep1 > 0.0, slot1.astype(jnp.int32), trash_j)
    lane_m = lax.broadcasted_iota(jnp.int32, (bt, 128), 1)

    def meta_rows(smv, ymv):
        return jnp.where(lane_m == 0, tgi,
                         jnp.where(lane_m == 1, smv,
                                   jnp.where(lane_m == 2, ymv, 0)))

    p0_ref[0] = p0
    p1_ref[0] = p1
    sm0_ref[0] = meta_rows(smv0, ym0)
    sm1_ref[0] = meta_rows(smv1, ym1)
    wr0_ref[0] = weff0
    wr1_ref[0] = weff1
    carry_ref[0:1, 0:1] = carryk + jnp.sum(ktot, axis=0, keepdims=True)


def _run_pos(e0, e1, r0, r1, w0, w1, counts, e, cap, cap2, ytrash, atr):
    nb, bt, _ = e0.shape
    tril = jnp.tril(jnp.ones((bt, bt), jnp.bfloat16), k=-1)
    tok_spec = pl.BlockSpec((1, bt, 1), lambda i: (i, 0, 0))
    meta_spec = pl.BlockSpec((1, bt, 128), lambda i: (i, 0, 0))
    ispec = [tok_spec] * 6 + [pl.BlockSpec((8, 128), lambda i: (0, 0)),
                              pl.BlockSpec((bt, bt), lambda i: (0, 0))]
    tok_i32 = jax.ShapeDtypeStruct((nb, bt, 1), jnp.int32)
    tok_f32 = jax.ShapeDtypeStruct((nb, bt, 1), jnp.float32)
    meta_i32 = jax.ShapeDtypeStruct((nb, bt, 128), jnp.int32)
    return pl.pallas_call(
        functools.partial(_pos_body, e, cap, cap2, bt, ytrash, atr),
        grid=(nb,),
        in_specs=ispec,
        out_specs=[tok_spec, tok_spec, meta_spec, meta_spec, tok_spec,
                   tok_spec, pl.BlockSpec((e, 128), lambda i: (0, 0))],
        out_shape=(tok_i32, tok_i32, meta_i32, meta_i32,
                   tok_f32, tok_f32,
                   jax.ShapeDtypeStruct((e, 128), jnp.int32)),
        scratch_shapes=[pltpu.VMEM((8, 128), jnp.float32)],
        compiler_params=pltpu.CompilerParams(
            dimension_semantics=("arbitrary",)),
    )(e0, e1, r0, r1, w0, w1, counts, tril)


def _make_listbuild(a):
    n_tok = a // _NW
    nch = n_tok // 64
    mesh = plsc.VectorSubcoreMesh(core_axis_name="c", subcore_axis_name="s")

    @functools.partial(
        pl.kernel, mesh=mesh,
        out_type=jax.ShapeDtypeStruct((_LMAX, 128), jnp.int32),
        scratch_types=[
            pltpu.VMEM((nch, 64), jnp.int32),
            pltpu.VMEM((64, 128), jnp.int32),
            pltpu.VMEM((64, 128), jnp.int32),
            pltpu.SemaphoreType.DMA,
            pltpu.SemaphoreType.DMA,
            pltpu.SemaphoreType.DMA,
            pltpu.SemaphoreType.DMA,
        ],
    )
    def listbuild(p0_hbm, p1_hbm, m0_hbm, m1_hbm,
                  list_hbm, p_v, rowa, rowb, sla, slb, sea, seb):
        wid = lax.axis_index("s") * _NC + lax.axis_index("c")
        rows = (rowa, rowb)
        sl = (sla, slb)
        sems = (sea, seb)
        for k, (pk, mk) in enumerate(((p0_hbm, m0_hbm), (p1_hbm, m1_hbm))):
            pltpu.sync_copy(pk.at[wid], p_v)
            pltpu.async_copy(mk.at[wid, 0], rows[0], sl[0])
            if nch > 1:
                pltpu.async_copy(mk.at[wid, 1], rows[1], sl[1])
            for ch in range(nch):
                b = ch % 2
                pltpu.make_async_copy(mk.at[wid, ch], rows[b], sl[b]).wait()
                pltpu.async_copy(rows[b], list_hbm.at[p_v.at[ch]], sems[b])
                pltpu.make_async_copy(rows[b], list_hbm.at[p_v.at[ch]],
                                      sems[b]).wait()
                if ch + 2 < nch:
                    pltpu.async_copy(mk.at[wid, ch + 2], rows[b], sl[b])

    return listbuild


def _run_extract(clist):
    lm = clist.shape[0]

    def body(cl_ref, tok_ref, slot_ref, ydst_ref):
        blkv = cl_ref[...]
        tok_ref[...] = blkv[:, 0:1]
        slot_ref[...] = blkv[:, 1:2]
        ydst_ref[...] = blkv[:, 2:3]

    col = jax.ShapeDtypeStruct((lm, 1), jnp.int32)
    return pl.pallas_call(
        body,
        grid=(1,),
        in_specs=[pl.BlockSpec((lm, 128), lambda i: (0, 0))],
        out_specs=[pl.BlockSpec((lm, 1), lambda i: (0, 0))] * 3,
        out_shape=(col, col, col),
    )(clist)


def _make_mover(src_rows, dst_rows, d):
    mesh = plsc.VectorSubcoreMesh(core_axis_name="c", subcore_axis_name="s")
    ng = _EPW // _NS

    @functools.partial(
        pl.kernel, mesh=mesh,
        out_type=jax.ShapeDtypeStruct((dst_rows, d), jnp.float32),
        scratch_types=[
            pltpu.VMEM((ng, _NS), jnp.int32),
            pltpu.VMEM((ng, _NS), jnp.int32),
            pltpu.VMEM((_NS, d), jnp.float32),
            pltpu.VMEM((_NS, d), jnp.float32),
            pltpu.SemaphoreType.DMA,
            pltpu.SemaphoreType.DMA,
            pltpu.SemaphoreType.DMA,
            pltpu.SemaphoreType.DMA,
        ],
    )
    def mover(si_hbm, di_hbm, src_hbm, dst_hbm, si_v, di_v,
              bufa, bufb, sga, sgb, ssa, ssb):
        wid = lax.axis_index("s") * _NC + lax.axis_index("c")
        pltpu.sync_copy(si_hbm.at[wid], si_v)
        pltpu.sync_copy(di_hbm.at[wid], di_v)
        bufs = (bufa, bufb)
        sg = (sga, sgb)
        ss = (ssa, ssb)
        for g in range(min(2, ng)):
            pltpu.async_copy(src_hbm.at[si_v.at[g]], bufs[g], sg[g])
        for g in range(ng):
            b = g % 2
            pltpu.make_async_copy(src_hbm.at[si_v.at[g]], bufs[b],
                                  sg[b]).wait()
            pltpu.async_copy(bufs[b], dst_hbm.at[di_v.at[g]], ss[b])
            pltpu.make_async_copy(bufs[b], dst_hbm.at[di_v.at[g]],
                                  ss[b]).wait()
            if g + 2 < ng:
                pltpu.async_copy(src_hbm.at[si_v.at[g + 2]], bufs[b], sg[b])

    return mover


def _mlp_body(e, nh, sinfo_ref, xb_ref, w1_ref, w3_ref, w2_ref, out_ref,
              xbf_ref):
    ei = pl.program_id(0)
    hstep = pl.program_id(1)
    lo = sinfo_ref[ei]
    hi = sinfo_ref[e + ei]
    w1b = w1_ref[0].astype(jnp.bfloat16)
    w3b = w3_ref[0].astype(jnp.bfloat16)
    w2b = w2_ref[0].astype(jnp.bfloat16)

    @pl.when(hstep == 0)
    def _():
        def cast_body(i, carry):
            rs = pl.ds(i * _RB, _RB)
            xbf_ref[rs, :] = xb_ref[0, rs, :].astype(jnp.bfloat16)
            return carry
        lax.fori_loop(lo, hi, cast_body, 0)

    def blk_body(i, carry):
        rs = pl.ds(i * _RB, _RB)
        xr = xbf_ref[rs, :]
        aa = jnp.dot(xr, w1b, preferred_element_type=jnp.float32)
        bb = jnp.dot(xr, w3b, preferred_element_type=jnp.float32)
        g = aa * jax.nn.sigmoid(aa) * bb
        cc = jnp.dot(g.astype(jnp.bfloat16), w2b,
                     preferred_element_type=jnp.float32)
        prev = out_ref[0, rs, :]
        out_ref[0, rs, :] = jnp.where(hstep == 0, cc, prev + cc)
        return carry

    lax.fori_loop(lo, hi, blk_body, 0)


def _run_mlp(xbuf, w1, w3, w2, sinfo):
    e, cap2, d = xbuf.shape
    h = w1.shape[2]
    bh = 256 if h % 256 == 0 else h
    nh = h // bh
    grid_spec = pltpu.PrefetchScalarGridSpec(
        num_scalar_prefetch=1,
        grid=(e, nh),
        in_specs=[
            pl.BlockSpec((1, cap2, d), lambda i, j, *_: (i, 0, 0)),
            pl.BlockSpec((1, d, bh), lambda i, j, *_: (i, 0, j)),
            pl.BlockSpec((1, d, bh), lambda i, j, *_: (i, 0, j)),
            pl.BlockSpec((1, bh, d), lambda i, j, *_: (i, j, 0)),
        ],
        out_specs=pl.BlockSpec((1, cap2, d), lambda i, j, *_: (i, 0, 0)),
        scratch_shapes=[pltpu.VMEM((cap2, d), jnp.bfloat16)],
    )
    return pl.pallas_call(
        functools.partial(_mlp_body, e, nh),
        grid_spec=grid_spec,
        out_shape=jax.ShapeDtypeStruct((e, cap2, d), jnp.float32),
        compiler_params=pltpu.CompilerParams(
            dimension_semantics=("arbitrary", "arbitrary")),
    )(sinfo, xbuf, w1, w3, w2)


def _fin_body(ya_ref, yb_ref, w0_ref, w1_ref, y_ref):
    w0 = w0_ref[0]
    w1 = w1_ref[0]
    ca = jnp.where(w0 > 0.0, w0 * ya_ref[...], 0.0)
    cb = jnp.where(w1 > 0.0, w1 * yb_ref[...], 0.0)
    y_ref[...] = ca + cb


def _run_final(yab, wr0, wr1, a, d, atr):
    nb, bt, _ = wr0.shape
    off = atr // bt
    tok_spec = pl.BlockSpec((1, bt, 1), lambda i: (i, 0, 0))
    return pl.pallas_call(
        _fin_body,
        grid=(nb,),
        in_specs=[
            pl.BlockSpec((bt, d), lambda i: (i, 0)),
            pl.BlockSpec((bt, d), lambda i: (off + i, 0)),
            tok_spec, tok_spec,
        ],
        out_specs=pl.BlockSpec((bt, d), lambda i: (i, 0)),
        out_shape=jax.ShapeDtypeStruct((a, d), jnp.float32),
    )(yab, yab, wr0, wr1)


def kernel(x, router_w, w1, w2, w3):
    a, d = x.shape
    e = router_w.shape[1]
    cap = max(1, int(math.ceil(1.25 * a * _TOPK / e)))
    cap2 = cap + 8
    n_slots = e * cap2
    atr = a + 1024
    ryab = 2 * atr
    ytrash = ryab - 1024

    e0, e1, r0, r1, w0, w1r_, counts = _run_router(x, router_w)
    p0, p1, m0, m1, wr0, wr1, blk = _run_pos(
        e0, e1, r0, r1, w0, w1r_, counts, e, cap, cap2, ytrash, atr)
    sinfo = jnp.concatenate([blk[:, 0], blk[:, 1]])

    n_tok = a // _NW
    nch = n_tok // 64
    clist = _make_listbuild(a)(
        p0.reshape(_NW, nch, 64), p1.reshape(_NW, nch, 64),
        m0.reshape(_NW, nch, 64, 128), m1.reshape(_NW, nch, 64, 128))

    ltok, lslot, lydst = _run_extract(clist)
    ng = _EPW // _NS
    ltok = ltok.reshape(_NW, ng, _NS)
    lslot = lslot.reshape(_NW, ng, _NS)
    lydst = lydst.reshape(_NW, ng, _NS)
    xbuf = _make_mover(a, n_slots, d)(ltok, lslot, x)

    yslots = _run_mlp(xbuf.reshape(e, cap2, d), w1, w3, w2, sinfo)

    yab = _make_mover(n_slots, ryab, d)(
        lslot, lydst, yslots.reshape(n_slots, d))

    return _run_final(yab, wr0, wr1, a, d, atr)
